# baseline (speedup 1.0000x reference)
import jax
import jax.numpy as jnp
from jax import lax
from jax.experimental import pallas as pl
from jax.experimental.pallas import tpu as pltpu
from jax.experimental.pallas import tpu_sc as plsc

_B = 64
_T = 8192
_C = 16
_NW = 32
_CH = 1024
_SLW = _CH + 256
_CHUNKS_PB = _T // _CH
_TASKS = _B * _CHUNKS_PB
_TASKS_PW = _TASKS // _NW


def _task_params(wid, k):
    task = wid * _TASKS_PW + k
    b = task // _CHUNKS_PB
    cs = pl.multiple_of((task % _CHUNKS_PB) * _CH, _CH)
    ss = pl.multiple_of(jnp.clip(cs - 128, 0, _T - _SLW), 128)
    return b, cs, ss


def _compute_task(wid, k, sig_v, out_v, wcs, bias):
    _, cs, ss = _task_params(wid, k)
    lb = cs - ss

    def blk(i, carry):
        tl = lb + i * 16
        acc = sig_v[0, pl.ds(tl, 16)] * wcs[0]
        for c in range(1, _C):
            acc = acc + sig_v[c, pl.ds(tl, 16)] * wcs[c]
        e = jnp.exp((acc + bias) * 2.0)
        d = 1.0 - 2.0 / (e + 1.0)

        tvec = (lax.iota(jnp.int32, 16) + (cs + i * 16)).astype(jnp.float32)
        x = tvec + d
        xi = x.astype(jnp.int32)
        xif = xi.astype(jnp.float32)
        x0u = jnp.where(xif > x, xi - 1, xi)
        x1u = x0u + 1
        x0c = jnp.clip(x0u, 0, _T - 1)
        x1c = jnp.clip(x1u, 0, _T - 1)
        w0 = x1c.astype(jnp.float32) - x
        w1 = x - x0c.astype(jnp.float32)
        pos0 = x0c - ss
        pos1 = x1c - ss

        o0 = i * 16
        for c in range(_C):
            cvec = jnp.full((16,), c, jnp.int32)
            v0 = plsc.load_gather(sig_v, [cvec, pos0])
            v1 = plsc.load_gather(sig_v, [cvec, pos1])
            out_v[c, pl.ds(o0, 16)] = w0 * v0 + w1 * v1
        return carry

    lax.fori_loop(0, _CH // 16, blk, 0)


def _sc_body(xt_ref, wcb_ref, out_hbm,
             sig0, sig1, out0, out1, wc_v,
             sin0, sin1, sout0, sout1, swc):
    wid = lax.axis_index("s") * 2 + lax.axis_index("c")

    pltpu.async_copy(wcb_ref, wc_v, swc).wait()
    wcs = [wc_v[c, :] for c in range(_C)]
    bias = wc_v[_C, :]

    sigs = (sig0, sig1)
    outs = (out0, out1)
    sins = (sin0, sin1)
    souts = (sout0, sout1)

    def in_copy(k, u):
        b, _, ss = _task_params(wid, k)
        return pltpu.make_async_copy(
            xt_ref.at[b, :, pl.ds(ss, _SLW)], sigs[u], sins[u])

    def out_copy(k, u):
        b, cs, _ = _task_params(wid, k)
        return pltpu.make_async_copy(
            outs[u], out_hbm.at[b, :, pl.ds(cs, _CH)], souts[u])

    def outer(j, carry):
        for u in (0, 1):
            k = j * 2 + u
            c = in_copy(k, u)
            c.start()
            c.wait()
            _compute_task(wid, k, sigs[u], outs[u], wcs, bias)
            oc = out_copy(k, u)
            oc.start()
            oc.wait()
        return carry

    lax.fori_loop(0, _TASKS_PW // 2, outer, 0)


def kernel(X, Wc, b):
    B, T, C = X.shape
    XT = X.transpose(0, 2, 1)
    wcb = jnp.zeros((24, 16), jnp.float32)
    wcb = wcb.at[:C, :].set(jnp.broadcast_to(Wc, (C, 16)))
    wcb = wcb.at[C, :].set(b[0])

    mesh = plsc.VectorSubcoreMesh(core_axis_name="c", subcore_axis_name="s")
    sc = pl.kernel(
        _sc_body,
        mesh=mesh,
        compiler_params=pltpu.CompilerParams(needs_layout_passes=False),
        out_type=jax.ShapeDtypeStruct((B, C, T), jnp.float32),
        scratch_types=[
            pltpu.VMEM((_C, _SLW), jnp.float32),
            pltpu.VMEM((_C, _SLW), jnp.float32),
            pltpu.VMEM((_C, _CH), jnp.float32),
            pltpu.VMEM((_C, _CH), jnp.float32),
            pltpu.VMEM((24, 16), jnp.float32),
            pltpu.SemaphoreType.DMA,
            pltpu.SemaphoreType.DMA,
            pltpu.SemaphoreType.DMA,
            pltpu.SemaphoreType.DMA,
            pltpu.SemaphoreType.DMA,
        ],
    )
    out = sc(XT, wcb)
    return out.transpose(0, 2, 1)


# speedup vs baseline: 1.1746x; 1.1746x over previous
import jax
import jax.numpy as jnp
from jax import lax
from jax.experimental import pallas as pl
from jax.experimental.pallas import tpu as pltpu
from jax.experimental.pallas import tpu_sc as plsc

_B = 64
_T = 8192
_C = 16
_NW = 32
_CH = 1024
_SLW = _CH + 256
_CHUNKS_PB = _T // _CH
_TASKS = _B * _CHUNKS_PB
_TASKS_PW = _TASKS // _NW


def _task_params(wid, k):
    task = wid * _TASKS_PW + k
    b = task // _CHUNKS_PB
    cs = pl.multiple_of((task % _CHUNKS_PB) * _CH, _CH)
    ss = pl.multiple_of(jnp.clip(cs - 128, 0, _T - _SLW), 128)
    return b, cs, ss


def _compute_task(wid, k, sig_v, out_v, wcs, bias):
    _, cs, ss = _task_params(wid, k)
    lb = cs - ss

    def blk(i, carry):
        tl = lb + i * 16
        acc = sig_v[0, pl.ds(tl, 16)] * wcs[0]
        for c in range(1, _C):
            acc = acc + sig_v[c, pl.ds(tl, 16)] * wcs[c]
        e = jnp.exp((acc + bias) * 2.0)
        d = 1.0 - 2.0 / (e + 1.0)

        tvec = (lax.iota(jnp.int32, 16) + (cs + i * 16)).astype(jnp.float32)
        x = tvec + d
        xi = x.astype(jnp.int32)
        xif = xi.astype(jnp.float32)
        x0u = jnp.where(xif > x, xi - 1, xi)
        x1u = x0u + 1
        x0c = jnp.clip(x0u, 0, _T - 1)
        x1c = jnp.clip(x1u, 0, _T - 1)
        w0 = x1c.astype(jnp.float32) - x
        w1 = x - x0c.astype(jnp.float32)
        pos0 = x0c - ss
        pos1 = x1c - ss

        o0 = i * 16
        for c in range(_C):
            cvec = jnp.full((16,), c, jnp.int32)
            v0 = plsc.load_gather(sig_v, [cvec, pos0])
            v1 = plsc.load_gather(sig_v, [cvec, pos1])
            out_v[c, pl.ds(o0, 16)] = w0 * v0 + w1 * v1
        return carry

    lax.fori_loop(0, _CH // 16, blk, 0)


def _sc_body(xt_ref, wcb_ref, out_hbm,
             sig0, sig1, out0, out1, wc_v,
             sin0, sin1, sout0, sout1, swc):
    wid = lax.axis_index("s") * 2 + lax.axis_index("c")

    pltpu.async_copy(wcb_ref, wc_v, swc).wait()
    wcs = [wc_v[c, :] for c in range(_C)]
    bias = wc_v[_C, :]

    sigs = (sig0, sig1)
    outs = (out0, out1)
    sins = (sin0, sin1)
    souts = (sout0, sout1)

    def in_copy(k, u):
        b, _, ss = _task_params(wid, k)
        return pltpu.make_async_copy(
            xt_ref.at[b, :, pl.ds(ss, _SLW)], sigs[u], sins[u])

    def out_copy(k, u):
        b, cs, _ = _task_params(wid, k)
        return pltpu.make_async_copy(
            outs[u], out_hbm.at[b, :, pl.ds(cs, _CH)], souts[u])

    in_copy(0, 0).start()

    def outer(j, carry):
        for u in (0, 1):
            k = j * 2 + u
            in_copy(k, u).wait()
            if u == 0:
                in_copy(k + 1, 1).start()
            else:
                @pl.when(j < _TASKS_PW // 2 - 1)
                def _():
                    in_copy(k + 1, 0).start()

            @pl.when(j > 0)
            def _():
                out_copy(k - 2, u).wait()

            _compute_task(wid, k, sigs[u], outs[u], wcs, bias)
            out_copy(k, u).start()
        return carry

    lax.fori_loop(0, _TASKS_PW // 2, outer, 0)
    out_copy(_TASKS_PW - 2, 0).wait()
    out_copy(_TASKS_PW - 1, 1).wait()


def kernel(X, Wc, b):
    B, T, C = X.shape
    XT = X.transpose(0, 2, 1)
    wcb = jnp.zeros((24, 16), jnp.float32)
    wcb = wcb.at[:C, :].set(jnp.broadcast_to(Wc, (C, 16)))
    wcb = wcb.at[C, :].set(b[0])

    mesh = plsc.VectorSubcoreMesh(core_axis_name="c", subcore_axis_name="s")
    sc = pl.kernel(
        _sc_body,
        mesh=mesh,
        compiler_params=pltpu.CompilerParams(needs_layout_passes=False),
        out_type=jax.ShapeDtypeStruct((B, C, T), jnp.float32),
        scratch_types=[
            pltpu.VMEM((_C, _SLW), jnp.float32),
            pltpu.VMEM((_C, _SLW), jnp.float32),
            pltpu.VMEM((_C, _CH), jnp.float32),
            pltpu.VMEM((_C, _CH), jnp.float32),
            pltpu.VMEM((24, 16), jnp.float32),
            pltpu.SemaphoreType.DMA,
            pltpu.SemaphoreType.DMA,
            pltpu.SemaphoreType.DMA,
            pltpu.SemaphoreType.DMA,
            pltpu.SemaphoreType.DMA,
        ],
    )
    out = sc(XT, wcb)
    return out.transpose(0, 2, 1)
